# Initial kernel scaffold; baseline (speedup 1.0000x reference)
import jax
import jax.numpy as jnp
from jax.experimental import pallas as pl


def kernel(x, conv_w, conv_b):
    raise NotImplementedError("write your pallas kernel here")


# speedup vs baseline: 188.0288x; 188.0288x over previous
import functools

import jax
import jax.numpy as jnp
from jax.experimental import pallas as pl
from jax.experimental.pallas import tpu as pltpu

K = 32
C = 384
NEG_INF = float("-inf")


def _topk_weights_body(cf_ref, w_ref, wc_ref):
    vals = cf_ref[...]
    B = vals.shape[0]
    iota = jax.lax.broadcasted_iota(jnp.int32, (B, C), 1)
    wc = jnp.zeros((B, C), jnp.float32)
    for t in range(K):
        m = jnp.max(vals, axis=1, keepdims=True)
        ismax = vals >= m
        first = jnp.min(jnp.where(ismax, iota, C), axis=1, keepdims=True)
        onehot = iota == first
        wc = wc + jnp.where(onehot, w_ref[t], 0.0)
        vals = jnp.where(onehot, NEG_INF, vals)
    wc_ref[...] = wc


def _attend_body(wc_ref, bias_ref, x_ref, o_ref):
    xb = x_ref[0]
    wcb = wc_ref[0]
    logits = jax.lax.dot_general(
        wcb, xb, (((1,), (0,)), ((), ())),
        preferred_element_type=jnp.float32)
    att = jax.nn.sigmoid(logits + bias_ref[0, 0])
    o_ref[0] = xb * att


def kernel(x, conv_w, conv_b):
    B, C_, H, W = x.shape
    S_TOT = H * W
    S = 3584
    n_s = S_TOT // S

    cf = x[:, :, H // 2, W // 2]
    w = conv_w[0, :, 0, 0]
    bias = conv_b.reshape(1, 1)

    wc = pl.pallas_call(
        _topk_weights_body,
        out_shape=jax.ShapeDtypeStruct((B, C_), jnp.float32),
        in_specs=[
            pl.BlockSpec((B, C_), lambda: (0, 0)),
            pl.BlockSpec(memory_space=pltpu.SMEM),
        ],
        out_specs=pl.BlockSpec((B, C_), lambda: (0, 0)),
    )(cf, w)

    xf = x.reshape(B, C_, S_TOT)
    wc3 = wc.reshape(B, 1, C_)
    out = pl.pallas_call(
        _attend_body,
        grid=(B, n_s),
        out_shape=jax.ShapeDtypeStruct((B, C_, S_TOT), jnp.float32),
        in_specs=[
            pl.BlockSpec((1, 1, C_), lambda b, s: (b, 0, 0)),
            pl.BlockSpec(memory_space=pltpu.SMEM),
            pl.BlockSpec((1, C_, S), lambda b, s: (b, 0, s)),
        ],
        out_specs=pl.BlockSpec((1, C_, S), lambda b, s: (b, 0, s)),
        compiler_params=pltpu.CompilerParams(
            dimension_semantics=("parallel", "parallel")),
    )(wc3, bias, xf)
    return out.reshape(B, C_, H, W)
